# baseline (speedup 1.0000x reference)
import functools

import jax
import jax.numpy as jnp
from jax import lax
from jax.experimental import pallas as pl
from jax.experimental.pallas import tpu as pltpu
from jax.experimental.pallas import tpu_sc as plsc

B = 16384
D = 64
W = 1000000
NC = 2
NS = 16
NW = NC * NS
BPW = B // NW
G = 16
NG = BPW // G
RF = 2
NR = G // RF
LANES = 16
WIN = 128
NSLOT = 8
HALF = G // 2
Y_LO = 0.0
Y_HI = 5.5


def _make_sc_kernel():
    mesh = plsc.VectorSubcoreMesh(core_axis_name="c", subcore_axis_name="s")

    @functools.partial(
        pl.kernel,
        mesh=mesh,
        compiler_params=pltpu.CompilerParams(
            needs_layout_passes=False, use_tc_tiling_on_sc=True),
        out_type=jax.ShapeDtypeStruct((B,), jnp.float32),
        scratch_types=[
            pltpu.VMEM((BPW,), jnp.int32),
            pltpu.VMEM((BPW,), jnp.int32),
            pltpu.VMEM((BPW,), jnp.int32),
            pltpu.VMEM((BPW,), jnp.int32),
            pltpu.VMEM((NSLOT, D, WIN), jnp.float32),
            pltpu.VMEM((2, D, RF * WIN), jnp.float32),
            pltpu.VMEM((NSLOT * WIN,), jnp.float32),
            pltpu.VMEM((G * WIN,), jnp.float32),
            pltpu.VMEM((BPW,), jnp.float32),
            pltpu.SemaphoreType.DMA,
            pltpu.SemaphoreType.DMA,
            pltpu.SemaphoreType.DMA,
        ],
    )
    def sc_kernel(su_hbm, si_hbm, fl_hbm, sl_hbm, uwt_hbm, iwt_hbm, ub_hbm,
                  ib_hbm, out_hbm, idx_u, idx_i, flv, slv, au, ai, bu, bi,
                  out_v, sem_u, sem_i, sem_b):
        wid = lax.axis_index("s") * NC + lax.axis_index("c")
        base = wid * BPW

        pltpu.sync_copy(su_hbm.at[wid], idx_u)
        pltpu.sync_copy(si_hbm.at[wid], idx_i)
        pltpu.sync_copy(fl_hbm.at[wid], flv)
        pltpu.sync_copy(sl_hbm.at[wid], slv)

        lanes = lax.iota(jnp.int32, LANES)

        def fire_user(k, vu16, fl16, sl16):
            lu = pl.multiple_of(vu16[k] & -WIN, WIN)
            sl = sl16[k]
            cw = pltpu.make_async_copy(
                uwt_hbm.at[:, pl.ds(lu, WIN)], au.at[sl], sem_u)
            cb = pltpu.make_async_copy(
                ub_hbm.at[pl.ds(lu, WIN)],
                bu.at[pl.ds(pl.multiple_of(sl * WIN, WIN), WIN)], sem_u)

            @pl.when(fl16[k] != 0)
            def _():
                cw.start()
                cb.start()
            return cw, cb

        def wait_user(k, fl16, copies):
            cw, cb = copies

            @pl.when(fl16[k] != 0)
            def _():
                cw.wait()
                cb.wait()

        def fire_items(p, li16):
            wcopies = []
            bcopies = []
            for k in range(RF):
                el = RF * p + k
                li = pl.multiple_of(li16[el] & -WIN, WIN)
                slab = p % 2
                dst = pl.ds(k * WIN, WIN)
                wcopies.append(pltpu.make_async_copy(
                    iwt_hbm.at[:, pl.ds(li, WIN)],
                    ai.at[slab].at[:, dst], sem_i))
                bcopies.append(pltpu.make_async_copy(
                    ib_hbm.at[pl.ds(li, WIN)],
                    bi.at[pl.ds(el * WIN, WIN)], sem_b))
            for c in wcopies + bcopies:
                c.start()
            return wcopies, bcopies

        def extract_round(p, acc, slu, colu, coli):
            slab = p % 2
            mp = (lanes // RF) == p
            part = jnp.zeros((LANES,), jnp.float32)
            for f in range(D):
                rowf = jnp.full((LANES,), f, jnp.int32)
                uv = plsc.load_gather(au, [slu, rowf, colu])
                iv = plsc.load_gather(ai.at[slab], [rowf, coli])
                part = part + uv * iv
            return acc + jnp.where(mp, part, 0.0)

        def group_body(g, _):
            e0 = g * G
            vu16 = idx_u[pl.ds(e0, LANES)]
            vi16 = idx_i[pl.ds(e0, LANES)]
            fl16 = flv[pl.ds(e0, LANES)]
            sl16 = slv[pl.ds(e0, LANES)]
            colu = vu16 & (WIN - 1)
            coli = (vi16 & (WIN - 1)) + (lanes % RF) * WIN

            acc = jnp.zeros((LANES,), jnp.float32)
            buv = jnp.zeros((LANES,), jnp.float32)
            colb_u = (vu16 & (WIN - 1)) + sl16 * WIN
            all_bias = []
            for h in range(2):
                ucopies = [fire_user(h * HALF + k, vu16, fl16, sl16)
                           for k in range(HALF)]
                inflight = []
                p0 = h * (NR // 2)
                w, b = fire_items(p0, vi16)
                inflight.append(w)
                all_bias += b
                for k in range(HALF):
                    wait_user(h * HALF + k, fl16, ucopies[k])
                for q in range(1, NR // 2):
                    w, b = fire_items(p0 + q, vi16)
                    inflight.append(w)
                    all_bias += b
                    for c in inflight.pop(0):
                        c.wait()
                    acc = extract_round(p0 + q - 1, acc, sl16, colu, coli)
                for c in inflight.pop(0):
                    c.wait()
                acc = extract_round(p0 + NR // 2 - 1, acc, sl16, colu, coli)
                hm = (lanes // HALF) == h
                buv = buv + jnp.where(hm, plsc.load_gather(bu, [colb_u]), 0.0)

            for c in all_bias:
                c.wait()

            colb_i = (vi16 & (WIN - 1)) + lanes * WIN
            biv = plsc.load_gather(bi, [colb_i])
            res = acc + buv + biv
            y = (Y_HI - Y_LO) / (1.0 + jnp.exp(-res)) + Y_LO
            out_v[pl.ds(e0, LANES)] = y
            return 0

        lax.fori_loop(0, NG, group_body, 0)

        pltpu.sync_copy(out_v, out_hbm.at[pl.ds(base, BPW)])

    return sc_kernel


_SC_KERNEL = _make_sc_kernel()


@jax.jit
def kernel(x, u_weight, i_weight, u_bias, i_bias):
    users = x[:, 0].astype(jnp.int32)
    items = x[:, 1].astype(jnp.int32)
    order = jnp.argsort(users)
    su = users[order]
    si = items[order]
    ublk = su & -WIN
    pos = jnp.arange(B, dtype=jnp.int32)
    same = jnp.concatenate([jnp.zeros((1,), jnp.bool_), ublk[1:] == ublk[:-1]])
    same = same & (pos % BPW != 0)
    flag = (~same).astype(jnp.int32)
    rank = jnp.cumsum(flag.reshape(NW, BPW), axis=1) - 1
    slot = (rank & (NSLOT - 1)).astype(jnp.int32)

    out_sorted = _SC_KERNEL(
        su.reshape(NW, BPW), si.reshape(NW, BPW),
        flag.reshape(NW, BPW), slot,
        u_weight.T, i_weight.T, u_bias.reshape(-1), i_bias.reshape(-1))
    return jnp.zeros((B,), jnp.float32).at[order].set(out_sorted)


# speedup vs baseline: 1.2551x; 1.2551x over previous
import functools

import jax
import jax.numpy as jnp
from jax import lax
from jax.experimental import pallas as pl
from jax.experimental.pallas import tpu as pltpu
from jax.experimental.pallas import tpu_sc as plsc

B = 16384
D = 64
W = 1000000
NC = 2
NS = 16
NW = NC * NS
BPW = B // NW
G = 16
NG = BPW // G
RF = 2
NR = G // RF
LANES = 16
WIN = 128
Y_LO = 0.0
Y_HI = 5.5


def _make_sc_kernel():
    mesh = plsc.VectorSubcoreMesh(core_axis_name="c", subcore_axis_name="s")

    @functools.partial(
        pl.kernel,
        mesh=mesh,
        compiler_params=pltpu.CompilerParams(
            needs_layout_passes=False, use_tc_tiling_on_sc=True),
        out_type=jax.ShapeDtypeStruct((B,), jnp.float32),
        scratch_types=[
            pltpu.VMEM((BPW,), jnp.int32),
            pltpu.VMEM((BPW,), jnp.int32),
            pltpu.VMEM((BPW // 128, 128), jnp.int32),
            pltpu.VMEM((BPW // 128, 128), jnp.int32),
            pltpu.VMEM((3, D, RF * WIN), jnp.float32),
            pltpu.VMEM((3, D, RF * WIN), jnp.float32),
            pltpu.VMEM((BPW,), jnp.float32),
            pltpu.VMEM((BPW,), jnp.float32),
            pltpu.VMEM((BPW,), jnp.float32),
            pltpu.SemaphoreType.DMA,
            pltpu.SemaphoreType.DMA,
        ],
    )
    def sc_kernel(users_hbm, items_hbm, users2_hbm, items2_hbm, uwt_hbm,
                  iwt_hbm, ub_hbm, ib_hbm, out_hbm, idx_u, idx_i, idx_u2,
                  idx_i2, au, ai, bval_u, bval_i, out_v, sem_w, sem_b):
        wid = lax.axis_index("s") * NC + lax.axis_index("c")
        base = wid * BPW

        pltpu.sync_copy(users_hbm.at[wid], idx_u)
        pltpu.sync_copy(items_hbm.at[wid], idx_i)
        pltpu.sync_copy(users2_hbm.at[wid], idx_u2)
        pltpu.sync_copy(items2_hbm.at[wid], idx_i2)

        bias_copies = []
        for j in range(BPW // 128):
            dst = pl.ds(j * 128, 128)
            bias_copies.append(pltpu.make_async_copy(
                ub_hbm.at[idx_u2.at[j]], bval_u.at[dst], sem_b))
            bias_copies.append(pltpu.make_async_copy(
                ib_hbm.at[idx_i2.at[j]], bval_i.at[dst], sem_b))
        for c in bias_copies:
            c.start()
        for c in bias_copies:
            c.wait()

        lanes = lax.iota(jnp.int32, LANES)

        def fire_round(p, lu16, li16):
            wcopies = []
            bcopies = []
            for k in range(RF):
                el = RF * p + k
                lu = pl.multiple_of(lu16[el], WIN)
                li = pl.multiple_of(li16[el], WIN)
                slab = p % 3
                dst = pl.ds(k * WIN, WIN)
                wcopies.append(pltpu.make_async_copy(
                    uwt_hbm.at[:, pl.ds(lu, WIN)],
                    au.at[slab].at[:, dst], sem_w))
                wcopies.append(pltpu.make_async_copy(
                    iwt_hbm.at[:, pl.ds(li, WIN)],
                    ai.at[slab].at[:, dst], sem_w))
            for c in wcopies + bcopies:
                c.start()
            return wcopies, bcopies

        def extract_round(p, acc, colu, coli):
            slab = p % 3
            mp = (lanes // RF) == p
            part = jnp.zeros((LANES,), jnp.float32)
            for f in range(D):
                rowf = jnp.full((LANES,), f, jnp.int32)
                uv = plsc.load_gather(au.at[slab], [rowf, colu])
                iv = plsc.load_gather(ai.at[slab], [rowf, coli])
                part = part + uv * iv
            return acc + jnp.where(mp, part, 0.0)

        def group_body(g, _):
            e0 = g * G
            vu16 = idx_u[pl.ds(e0, LANES)]
            vi16 = idx_i[pl.ds(e0, LANES)]
            lu16 = vu16 & -WIN
            li16 = vi16 & -WIN
            colu = (vu16 & (WIN - 1)) + (lanes % RF) * WIN
            coli = (vi16 & (WIN - 1)) + (lanes % RF) * WIN

            acc = jnp.zeros((LANES,), jnp.float32)
            all_bias = []
            inflight = []
            for p in range(2):
                w, b = fire_round(p, lu16, li16)
                inflight.append(w)
                all_bias += b
            for p in range(2, NR):
                w, b = fire_round(p, lu16, li16)
                inflight.append(w)
                all_bias += b
                for c in inflight.pop(0):
                    c.wait()
                acc = extract_round(p - 2, acc, colu, coli)
            for p in range(NR - 2, NR):
                for c in inflight.pop(0):
                    c.wait()
                acc = extract_round(p, acc, colu, coli)

            res = acc + bval_u[pl.ds(e0, LANES)] + bval_i[pl.ds(e0, LANES)]
            y = (Y_HI - Y_LO) / (1.0 + jnp.exp(-res)) + Y_LO
            out_v[pl.ds(e0, LANES)] = y
            return 0

        lax.fori_loop(0, NG, group_body, 0)

        pltpu.sync_copy(out_v, out_hbm.at[pl.ds(base, BPW)])

    return sc_kernel


_SC_KERNEL = _make_sc_kernel()


@jax.jit
def kernel(x, u_weight, i_weight, u_bias, i_bias):
    users = x[:, 0].astype(jnp.int32)
    items = x[:, 1].astype(jnp.int32)
    return _SC_KERNEL(users.reshape(NW, BPW), items.reshape(NW, BPW),
                      users.reshape(NW, BPW // 128, 128),
                      items.reshape(NW, BPW // 128, 128),
                      u_weight.T, i_weight.T,
                      u_bias.reshape(-1), i_bias.reshape(-1))


# speedup vs baseline: 1.2824x; 1.0217x over previous
import functools

import jax
import jax.numpy as jnp
from jax import lax
from jax.experimental import pallas as pl
from jax.experimental.pallas import tpu as pltpu
from jax.experimental.pallas import tpu_sc as plsc

B = 16384
D = 64
W = 1000000
NC = 2
NS = 16
NW = NC * NS
BPW = B // NW
G = 16
NG = BPW // G
RF = 2
NR = G // RF
LANES = 16
WIN = 128
Y_LO = 0.0
Y_HI = 5.5


def _make_sc_kernel():
    mesh = plsc.VectorSubcoreMesh(core_axis_name="c", subcore_axis_name="s")

    @functools.partial(
        pl.kernel,
        mesh=mesh,
        compiler_params=pltpu.CompilerParams(
            needs_layout_passes=False, use_tc_tiling_on_sc=True),
        out_type=jax.ShapeDtypeStruct((B,), jnp.float32),
        scratch_types=[
            pltpu.VMEM((BPW + G,), jnp.int32),
            pltpu.VMEM((BPW + G,), jnp.int32),
            pltpu.VMEM((BPW // 128, 128), jnp.int32),
            pltpu.VMEM((BPW // 128, 128), jnp.int32),
            pltpu.VMEM((2, D, RF * WIN), jnp.float32),
            pltpu.VMEM((2, D, RF * WIN), jnp.float32),
            pltpu.VMEM((BPW,), jnp.float32),
            pltpu.VMEM((BPW,), jnp.float32),
            pltpu.VMEM((BPW,), jnp.float32),
            pltpu.SemaphoreType.DMA,
            pltpu.SemaphoreType.DMA,
        ],
    )
    def sc_kernel(users_hbm, items_hbm, users2_hbm, items2_hbm, uwt_hbm,
                  iwt_hbm, ub_hbm, ib_hbm, out_hbm, idx_u, idx_i, idx_u2,
                  idx_i2, au, ai, bval_u, bval_i, out_v, sem_w, sem_b):
        wid = lax.axis_index("s") * NC + lax.axis_index("c")
        base = wid * BPW

        pltpu.sync_copy(users_hbm.at[wid], idx_u.at[pl.ds(0, BPW)])
        pltpu.sync_copy(items_hbm.at[wid], idx_i.at[pl.ds(0, BPW)])
        pltpu.sync_copy(users2_hbm.at[wid], idx_u2)
        pltpu.sync_copy(items2_hbm.at[wid], idx_i2)

        bias_copies = []
        for j in range(BPW // 128):
            dst = pl.ds(j * 128, 128)
            bias_copies.append(pltpu.make_async_copy(
                ub_hbm.at[idx_u2.at[j]], bval_u.at[dst], sem_b))
            bias_copies.append(pltpu.make_async_copy(
                ib_hbm.at[idx_i2.at[j]], bval_i.at[dst], sem_b))
        for c in bias_copies:
            c.start()
        for c in bias_copies:
            c.wait()

        lanes = lax.iota(jnp.int32, LANES)

        def round_copies(p, lu16, li16):
            slab = p % 2
            copies = []
            for k in range(RF):
                el = (RF * p + k) % G
                lu = pl.multiple_of(lu16[el] & -WIN, WIN)
                li = pl.multiple_of(li16[el] & -WIN, WIN)
                dst = pl.ds(k * WIN, WIN)
                copies.append(pltpu.make_async_copy(
                    uwt_hbm.at[:, pl.ds(lu, WIN)],
                    au.at[slab].at[:, dst], sem_w))
                copies.append(pltpu.make_async_copy(
                    iwt_hbm.at[:, pl.ds(li, WIN)],
                    ai.at[slab].at[:, dst], sem_w))
            return copies

        def extract_round(p, acc, colu, coli):
            slab = p % 2
            mp = (lanes // RF) == p
            part = jnp.zeros((LANES,), jnp.float32)
            for f in range(D):
                rowf = jnp.full((LANES,), f, jnp.int32)
                uv = plsc.load_gather(au.at[slab], [rowf, colu])
                iv = plsc.load_gather(ai.at[slab], [rowf, coli])
                part = part + uv * iv
            return acc + jnp.where(mp, part, 0.0)

        vu0 = idx_u[pl.ds(0, LANES)]
        vi0 = idx_i[pl.ds(0, LANES)]
        for c in round_copies(0, vu0, vi0):
            c.start()

        def group_body(g, _):
            e0 = g * G
            vu16 = idx_u[pl.ds(e0, LANES)]
            vi16 = idx_i[pl.ds(e0, LANES)]
            vu16n = idx_u[pl.ds(e0 + G, LANES)]
            vi16n = idx_i[pl.ds(e0 + G, LANES)]
            colu = (vu16 & (WIN - 1)) + (lanes % RF) * WIN
            coli = (vi16 & (WIN - 1)) + (lanes % RF) * WIN

            acc = jnp.zeros((LANES,), jnp.float32)
            for p in range(NR):
                if p < NR - 1:
                    for c in round_copies(p + 1, vu16, vi16):
                        c.start()
                else:
                    nxt = round_copies(0, vu16n, vi16n)

                    @pl.when(g < NG - 1)
                    def _():
                        for c in nxt:
                            c.start()
                for c in round_copies(p, vu16, vi16):
                    c.wait()
                acc = extract_round(p, acc, colu, coli)

            res = acc + bval_u[pl.ds(e0, LANES)] + bval_i[pl.ds(e0, LANES)]
            y = (Y_HI - Y_LO) / (1.0 + jnp.exp(-res)) + Y_LO
            out_v[pl.ds(e0, LANES)] = y
            return 0

        lax.fori_loop(0, NG, group_body, 0)

        pltpu.sync_copy(out_v, out_hbm.at[pl.ds(base, BPW)])

    return sc_kernel


_SC_KERNEL = _make_sc_kernel()


@jax.jit
def kernel(x, u_weight, i_weight, u_bias, i_bias):
    users = x[:, 0].astype(jnp.int32)
    items = x[:, 1].astype(jnp.int32)
    return _SC_KERNEL(users.reshape(NW, BPW), items.reshape(NW, BPW),
                      users.reshape(NW, BPW // 128, 128),
                      items.reshape(NW, BPW // 128, 128),
                      u_weight.T, i_weight.T,
                      u_bias.reshape(-1), i_bias.reshape(-1))


# speedup vs baseline: 1.2910x; 1.0067x over previous
import functools

import jax
import jax.numpy as jnp
from jax import lax
from jax.experimental import pallas as pl
from jax.experimental.pallas import tpu as pltpu
from jax.experimental.pallas import tpu_sc as plsc

B = 16384
D = 64
W = 1000000
NC = 2
NS = 16
NW = NC * NS
BPW = B // NW
G = 16
NG = BPW // G
RF = 2
NR = G // RF
LANES = 16
WIN = 128
Y_LO = 0.0
Y_HI = 5.5


def _make_sc_kernel():
    mesh = plsc.VectorSubcoreMesh(core_axis_name="c", subcore_axis_name="s")

    @functools.partial(
        pl.kernel,
        mesh=mesh,
        compiler_params=pltpu.CompilerParams(
            needs_layout_passes=False, use_tc_tiling_on_sc=True),
        out_type=jax.ShapeDtypeStruct((B,), jnp.float32),
        scratch_types=[
            pltpu.VMEM((BPW + G,), jnp.int32),
            pltpu.VMEM((BPW + G,), jnp.int32),
            pltpu.VMEM((BPW // 128, 128), jnp.int32),
            pltpu.VMEM((BPW // 128, 128), jnp.int32),
            pltpu.VMEM((2, D, RF * WIN), jnp.float32),
            pltpu.VMEM((2, D, RF * WIN), jnp.float32),
            pltpu.VMEM((BPW,), jnp.float32),
            pltpu.VMEM((BPW,), jnp.float32),
            pltpu.VMEM((BPW,), jnp.float32),
            pltpu.SemaphoreType.DMA,
            pltpu.SemaphoreType.DMA,
        ],
    )
    def sc_kernel(users_hbm, items_hbm, users2_hbm, items2_hbm, uwt_hbm,
                  iwt_hbm, ub_hbm, ib_hbm, out_hbm, idx_u, idx_i, idx_u2,
                  idx_i2, au, ai, bval_u, bval_i, out_v, sem_w, sem_b):
        wid = lax.axis_index("s") * NC + lax.axis_index("c")
        base = wid * BPW

        pltpu.sync_copy(users_hbm.at[wid], idx_u.at[pl.ds(0, BPW)])
        pltpu.sync_copy(items_hbm.at[wid], idx_i.at[pl.ds(0, BPW)])
        pltpu.sync_copy(users2_hbm.at[wid], idx_u2)
        pltpu.sync_copy(items2_hbm.at[wid], idx_i2)
        idx_u[pl.ds(BPW, G)] = jnp.zeros((G,), jnp.int32)
        idx_i[pl.ds(BPW, G)] = jnp.zeros((G,), jnp.int32)

        bias_copies = []
        for j in range(BPW // 128):
            dst = pl.ds(j * 128, 128)
            bias_copies.append(pltpu.make_async_copy(
                ub_hbm.at[idx_u2.at[j]], bval_u.at[dst], sem_b))
            bias_copies.append(pltpu.make_async_copy(
                ib_hbm.at[idx_i2.at[j]], bval_i.at[dst], sem_b))
        for c in bias_copies:
            c.start()
        for c in bias_copies:
            c.wait()

        lanes = lax.iota(jnp.int32, LANES)

        def round_copies(p, lu16, li16):
            slab = p % 2
            copies = []
            for k in range(RF):
                el = (RF * p + k) % G
                lu = pl.multiple_of(lu16[el] & -WIN, WIN)
                li = pl.multiple_of(li16[el] & -WIN, WIN)
                dst = pl.ds(k * WIN, WIN)
                copies.append(pltpu.make_async_copy(
                    uwt_hbm.at[:, pl.ds(lu, WIN)],
                    au.at[slab].at[:, dst], sem_w))
                copies.append(pltpu.make_async_copy(
                    iwt_hbm.at[:, pl.ds(li, WIN)],
                    ai.at[slab].at[:, dst], sem_w))
            return copies

        def extract_round(p, acc, colu, coli):
            slab = p % 2
            mp = (lanes // RF) == p
            part = jnp.zeros((LANES,), jnp.float32)
            for f in range(D):
                rowf = jnp.full((LANES,), f, jnp.int32)
                uv = plsc.load_gather(au.at[slab], [rowf, colu])
                iv = plsc.load_gather(ai.at[slab], [rowf, coli])
                part = part + uv * iv
            return acc + jnp.where(mp, part, 0.0)

        vu0 = idx_u[pl.ds(0, LANES)]
        vi0 = idx_i[pl.ds(0, LANES)]
        for c in round_copies(0, vu0, vi0):
            c.start()

        def group_body(g, _):
            e0 = g * G
            vu16 = idx_u[pl.ds(e0, LANES)]
            vi16 = idx_i[pl.ds(e0, LANES)]
            vu16n = idx_u[pl.ds(e0 + G, LANES)]
            vi16n = idx_i[pl.ds(e0 + G, LANES)]
            colu = (vu16 & (WIN - 1)) + (lanes % RF) * WIN
            coli = (vi16 & (WIN - 1)) + (lanes % RF) * WIN

            acc = jnp.zeros((LANES,), jnp.float32)
            for p in range(NR):
                if p < NR - 1:
                    for c in round_copies(p + 1, vu16, vi16):
                        c.start()
                else:
                    nxt = round_copies(0, vu16n, vi16n)

                    @pl.when(g < NG - 1)
                    def _():
                        for c in nxt:
                            c.start()
                for c in round_copies(p, vu16, vi16):
                    c.wait()
                acc = extract_round(p, acc, colu, coli)

            res = acc + bval_u[pl.ds(e0, LANES)] + bval_i[pl.ds(e0, LANES)]
            y = (Y_HI - Y_LO) / (1.0 + jnp.exp(-res)) + Y_LO
            out_v[pl.ds(e0, LANES)] = y
            return 0

        lax.fori_loop(0, NG, group_body, 0)

        pltpu.sync_copy(out_v, out_hbm.at[pl.ds(base, BPW)])

    return sc_kernel


_SC_KERNEL = _make_sc_kernel()


@jax.jit
def kernel(x, u_weight, i_weight, u_bias, i_bias):
    users = x[:, 0].astype(jnp.int32)
    items = x[:, 1].astype(jnp.int32)
    return _SC_KERNEL(users.reshape(NW, BPW), items.reshape(NW, BPW),
                      users.reshape(NW, BPW // 128, 128),
                      items.reshape(NW, BPW // 128, 128),
                      u_weight.T, i_weight.T,
                      u_bias.reshape(-1), i_bias.reshape(-1))
